# Initial kernel scaffold; baseline (speedup 1.0000x reference)
import jax
import jax.numpy as jnp
from jax.experimental import pallas as pl


def kernel(x_item, x_user, edge_index_ui, edge_index_iu, W_ui, b_ui, W_iu, b_iu, W_proj, b_proj):
    raise NotImplementedError("write your pallas kernel here")


# speedup vs baseline: 8.2683x; 8.2683x over previous
import functools

import jax
import jax.numpy as jnp
from jax import lax
from jax.experimental import pallas as pl
from jax.experimental.pallas import tpu as pltpu
from jax.experimental.pallas import tpu_sc as plsc

N = 10000
E = 320000
D = 128

NS = 16
NP = 10112
ROWS_PT = NP // NS
CHUNK = 128
IDXBLK = 8
NCHUNK = 160
EPT = CHUNK * NCHUNK
EPAD = EPT * NS

_mesh2 = plsc.VectorSubcoreMesh(core_axis_name="c", subcore_axis_name="s")
_mesh1 = plsc.VectorSubcoreMesh(core_axis_name="c", subcore_axis_name="s",
                                num_cores=1)


@functools.partial(
    pl.kernel,
    mesh=_mesh2,
    compiler_params=pltpu.CompilerParams(use_tc_tiling_on_sc=False),
    out_type=jax.ShapeDtypeStruct((2, NP, 16), jnp.float32),
    scratch_types=[
        pltpu.VMEM_SHARED((NP, 16), jnp.float32),
        pltpu.VMEM((IDXBLK, CHUNK), jnp.int32),
        pltpu.VMEM((CHUNK, 16), jnp.float32),
        pltpu.SemaphoreType.DMA,
    ],
)
def _deg_kernel(dst_hbm, zeros_hbm, deg_out, deg_sh, dst_v, ones_v, sem):
    c = lax.axis_index("c")
    s = lax.axis_index("s")
    pltpu.sync_copy(zeros_hbm, deg_sh.at[pl.ds(s * ROWS_PT, ROWS_PT)])

    @pl.loop(0, CHUNK)
    def _(j):
        ones_v.at[j][...] = jnp.full((16,), 1.0, jnp.float32)

    plsc.subcore_barrier()

    @pl.loop(0, NCHUNK, step=IDXBLK)
    def _(r):
        pltpu.sync_copy(dst_hbm.at[c, s, pl.ds(r, IDXBLK)], dst_v)

        @pl.loop(0, IDXBLK)
        def _(j):
            pltpu.sync_copy(ones_v, deg_sh.at[dst_v.at[j]], add=True)

    plsc.subcore_barrier()
    pltpu.sync_copy(deg_sh.at[pl.ds(s * ROWS_PT, ROWS_PT)],
                    deg_out.at[c, pl.ds(s * ROWS_PT, ROWS_PT)])


@functools.partial(
    pl.kernel,
    mesh=_mesh1,
    out_type=jax.ShapeDtypeStruct((NP, D), jnp.float32),
    scratch_types=[
        pltpu.VMEM_SHARED((NP, D), jnp.float32),
        pltpu.VMEM((IDXBLK, CHUNK), jnp.int32),
        pltpu.VMEM((IDXBLK, CHUNK), jnp.int32),
        pltpu.VMEM((CHUNK, D), jnp.float32),
        pltpu.SemaphoreType.DMA,
    ],
)
def _edge_kernel(y_hbm, src_hbm, dst_hbm, zeros_hbm, acc_out,
                 acc_sh, src_v, dst_v, rows_v, sem):
    s = lax.axis_index("s")
    pltpu.sync_copy(zeros_hbm, acc_sh.at[pl.ds(s * ROWS_PT, ROWS_PT)])
    plsc.subcore_barrier()

    @pl.loop(0, NCHUNK, step=IDXBLK)
    def _(r):
        pltpu.sync_copy(src_hbm.at[s, pl.ds(r, IDXBLK)], src_v)
        pltpu.sync_copy(dst_hbm.at[s, pl.ds(r, IDXBLK)], dst_v)

        @pl.loop(0, IDXBLK)
        def _(j):
            pltpu.async_copy(y_hbm.at[src_v.at[j]], rows_v, sem).wait()
            pltpu.sync_copy(rows_v, acc_sh.at[dst_v.at[j]], add=True)

    plsc.subcore_barrier()
    pltpu.sync_copy(acc_sh.at[pl.ds(s * ROWS_PT, ROWS_PT)],
                    acc_out.at[pl.ds(s * ROWS_PT, ROWS_PT)])


_BLK = 2528


def _xw_body(x1_ref, x2_ref, w1_ref, w2_ref, o1_ref, o2_ref):
    o1_ref[...] = jnp.dot(x1_ref[...], w1_ref[...],
                          preferred_element_type=jnp.float32)
    o2_ref[...] = jnp.dot(x2_ref[...], w2_ref[...],
                          preferred_element_type=jnp.float32)


def _tc_xw(x1, x2, w1, w2):
    return pl.pallas_call(
        _xw_body,
        grid=(NP // _BLK,),
        in_specs=[
            pl.BlockSpec((_BLK, D), lambda i: (i, 0)),
            pl.BlockSpec((_BLK, D), lambda i: (i, 0)),
            pl.BlockSpec((D, D), lambda i: (0, 0)),
            pl.BlockSpec((D, D), lambda i: (0, 0)),
        ],
        out_specs=[
            pl.BlockSpec((_BLK, D), lambda i: (i, 0)),
            pl.BlockSpec((_BLK, D), lambda i: (i, 0)),
        ],
        out_shape=[
            jax.ShapeDtypeStruct((NP, D), jnp.float32),
            jax.ShapeDtypeStruct((NP, D), jnp.float32),
        ],
    )(x1, x2, w1, w2)


def _scale_body(xw1_ref, xw2_ref, deg1_ref, deg2_ref, y1_ref, y2_ref):
    d1 = lax.rsqrt(deg1_ref[...][:, 0:1] + 1.0)
    d2 = lax.rsqrt(deg2_ref[...][:, 0:1] + 1.0)
    y1_ref[...] = xw1_ref[...] * d1
    y2_ref[...] = xw2_ref[...] * d2


def _tc_scale(xw1, xw2, deg1, deg2):
    return pl.pallas_call(
        _scale_body,
        grid=(NP // _BLK,),
        in_specs=[
            pl.BlockSpec((_BLK, D), lambda i: (i, 0)),
            pl.BlockSpec((_BLK, D), lambda i: (i, 0)),
            pl.BlockSpec((_BLK, 16), lambda i: (i, 0)),
            pl.BlockSpec((_BLK, 16), lambda i: (i, 0)),
        ],
        out_specs=[
            pl.BlockSpec((_BLK, D), lambda i: (i, 0)),
            pl.BlockSpec((_BLK, D), lambda i: (i, 0)),
        ],
        out_shape=[
            jax.ShapeDtypeStruct((NP, D), jnp.float32),
            jax.ShapeDtypeStruct((NP, D), jnp.float32),
        ],
    )(xw1, xw2, deg1, deg2)


def _final_body(acc1_ref, y1_ref, deg1_ref, b1_ref,
                acc2_ref, y2_ref, deg2_ref, b2_ref,
                wp_ref, bp_ref, o1_ref, o2_ref):
    for acc, y, deg, b, o in (
        (acc1_ref, y1_ref, deg1_ref, b1_ref, o1_ref),
        (acc2_ref, y2_ref, deg2_ref, b2_ref, o2_ref),
    ):
        d = lax.rsqrt(deg[...][:, 0:1] + 1.0)
        t = (acc[...] + y[...]) * d + b[...]
        t = jnp.maximum(t, 0.0)
        o[...] = jnp.dot(t, wp_ref[...],
                         preferred_element_type=jnp.float32) + bp_ref[...]


def _tc_final(acc1, y1, deg1, b1, acc2, y2, deg2, b2, wp, bp):
    return pl.pallas_call(
        _final_body,
        grid=(NP // _BLK,),
        in_specs=[
            pl.BlockSpec((_BLK, D), lambda i: (i, 0)),
            pl.BlockSpec((_BLK, D), lambda i: (i, 0)),
            pl.BlockSpec((_BLK, 16), lambda i: (i, 0)),
            pl.BlockSpec((1, D), lambda i: (0, 0)),
            pl.BlockSpec((_BLK, D), lambda i: (i, 0)),
            pl.BlockSpec((_BLK, D), lambda i: (i, 0)),
            pl.BlockSpec((_BLK, 16), lambda i: (i, 0)),
            pl.BlockSpec((1, D), lambda i: (0, 0)),
            pl.BlockSpec((D, D), lambda i: (0, 0)),
            pl.BlockSpec((1, D), lambda i: (0, 0)),
        ],
        out_specs=[
            pl.BlockSpec((_BLK, D), lambda i: (i, 0)),
            pl.BlockSpec((_BLK, D), lambda i: (i, 0)),
        ],
        out_shape=[
            jax.ShapeDtypeStruct((NP, D), jnp.float32),
            jax.ShapeDtypeStruct((NP, D), jnp.float32),
        ],
    )(acc1, y1, deg1, b1, acc2, y2, deg2, b2, wp, bp)


def _prep_edges(ei):
    pad = EPAD - E
    src = jnp.concatenate([ei[0].astype(jnp.int32),
                           jnp.full((pad,), N, jnp.int32)])
    dst = jnp.concatenate([ei[1].astype(jnp.int32),
                           N + (jnp.arange(pad, dtype=jnp.int32) % (NP - N))])
    return (src.reshape(NS, NCHUNK, CHUNK), dst.reshape(NS, NCHUNK, CHUNK))


def kernel(x_item, x_user, edge_index_ui, edge_index_iu,
           W_ui, b_ui, W_iu, b_iu, W_proj, b_proj):
    f32 = jnp.float32
    xi = jnp.pad(x_item.astype(f32), ((0, NP - N), (0, 0)))
    xu = jnp.pad(x_user.astype(f32), ((0, NP - N), (0, 0)))
    src0, dst0 = _prep_edges(edge_index_ui)
    src1, dst1 = _prep_edges(edge_index_iu)
    dst_both = jnp.stack([dst0, dst1])

    zeros16 = jnp.zeros((ROWS_PT, 16), f32)
    zerosD = jnp.zeros((ROWS_PT, D), f32)

    deg = _deg_kernel(dst_both, zeros16)
    xw_ui, xw_iu = _tc_xw(xi, xu, W_ui, W_iu)
    y_ui, y_iu = _tc_scale(xw_ui, xw_iu, deg[0], deg[1])
    acc_ui = _edge_kernel(y_ui, src0, dst0, zerosD)
    acc_iu = _edge_kernel(y_iu, src1, dst1, zerosD)
    out_item, out_user = _tc_final(
        acc_ui, y_ui, deg[0], b_ui.reshape(1, D),
        acc_iu, y_iu, deg[1], b_iu.reshape(1, D),
        W_proj, b_proj.reshape(1, D))
    return (out_item[:N], out_user[:N])


# speedup vs baseline: 9.2586x; 1.1198x over previous
import functools

import jax
import jax.numpy as jnp
from jax import lax
from jax.experimental import pallas as pl
from jax.experimental.pallas import tpu as pltpu
from jax.experimental.pallas import tpu_sc as plsc

N = 10000
E = 320000
D = 128

NS = 16
NP = 10112
ROWS_PT = NP // NS
CHUNK = 128
IDXBLK = 8
NCHUNK = 160
EPT = CHUNK * NCHUNK
EPAD = EPT * NS

_mesh2 = plsc.VectorSubcoreMesh(core_axis_name="c", subcore_axis_name="s")
_mesh1 = plsc.VectorSubcoreMesh(core_axis_name="c", subcore_axis_name="s",
                                num_cores=1)


@functools.partial(
    pl.kernel,
    mesh=_mesh2,
    compiler_params=pltpu.CompilerParams(use_tc_tiling_on_sc=False),
    out_type=jax.ShapeDtypeStruct((2, NP, 16), jnp.float32),
    scratch_types=[
        pltpu.VMEM_SHARED((NP, 16), jnp.float32),
        pltpu.VMEM((IDXBLK, CHUNK), jnp.int32),
        pltpu.VMEM((CHUNK, 16), jnp.float32),
        pltpu.SemaphoreType.DMA,
    ],
)
def _deg_kernel(dst_hbm, zeros_hbm, deg_out, deg_sh, dst_v, ones_v, sem):
    c = lax.axis_index("c")
    s = lax.axis_index("s")
    pltpu.sync_copy(zeros_hbm, deg_sh.at[pl.ds(s * ROWS_PT, ROWS_PT)])

    @pl.loop(0, CHUNK)
    def _(j):
        ones_v.at[j][...] = jnp.full((16,), 1.0, jnp.float32)

    plsc.subcore_barrier()

    @pl.loop(0, NCHUNK, step=IDXBLK)
    def _(r):
        pltpu.sync_copy(dst_hbm.at[c, s, pl.ds(r, IDXBLK)], dst_v)

        @pl.loop(0, IDXBLK)
        def _(j):
            pltpu.sync_copy(ones_v, deg_sh.at[dst_v.at[j]], add=True)

    plsc.subcore_barrier()
    pltpu.sync_copy(deg_sh.at[pl.ds(s * ROWS_PT, ROWS_PT)],
                    deg_out.at[c, pl.ds(s * ROWS_PT, ROWS_PT)])


@functools.partial(
    pl.kernel,
    mesh=_mesh1,
    out_type=jax.ShapeDtypeStruct((NP, D), jnp.float32),
    scratch_types=[
        pltpu.VMEM_SHARED((NP, D), jnp.float32),
        pltpu.VMEM((IDXBLK, CHUNK), jnp.int32),
        pltpu.VMEM((IDXBLK, CHUNK), jnp.int32),
        pltpu.VMEM((CHUNK, D), jnp.float32),
        pltpu.VMEM((CHUNK, D), jnp.float32),
        pltpu.SemaphoreType.DMA,
    ],
)
def _edge_kernel(y_hbm, src_hbm, dst_hbm, zeros_hbm, acc_out,
                 acc_sh, src_v, dst_v, rows0_v, rows1_v, gsem):
    s = lax.axis_index("s")
    pltpu.sync_copy(zeros_hbm, acc_sh.at[pl.ds(s * ROWS_PT, ROWS_PT)])
    plsc.subcore_barrier()
    rows = (rows0_v, rows1_v)

    @pl.loop(0, NCHUNK, step=IDXBLK)
    def _(r):
        pltpu.sync_copy(src_hbm.at[s, pl.ds(r, IDXBLK)], src_v)
        pltpu.sync_copy(dst_hbm.at[s, pl.ds(r, IDXBLK)], dst_v)
        cp = pltpu.async_copy(y_hbm.at[src_v.at[0]], rows[0], gsem)
        for k in range(IDXBLK):
            cp.wait()
            if k + 1 < IDXBLK:
                cp = pltpu.async_copy(y_hbm.at[src_v.at[k + 1]],
                                      rows[(k + 1) % 2], gsem)
            pltpu.sync_copy(rows[k % 2], acc_sh.at[dst_v.at[k]], add=True)

    plsc.subcore_barrier()
    pltpu.sync_copy(acc_sh.at[pl.ds(s * ROWS_PT, ROWS_PT)],
                    acc_out.at[pl.ds(s * ROWS_PT, ROWS_PT)])


_BLK = 2528


def _xw_body(x1_ref, x2_ref, w1_ref, w2_ref, o1_ref, o2_ref):
    o1_ref[...] = jnp.dot(x1_ref[...], w1_ref[...],
                          preferred_element_type=jnp.float32)
    o2_ref[...] = jnp.dot(x2_ref[...], w2_ref[...],
                          preferred_element_type=jnp.float32)


def _tc_xw(x1, x2, w1, w2):
    return pl.pallas_call(
        _xw_body,
        grid=(NP // _BLK,),
        in_specs=[
            pl.BlockSpec((_BLK, D), lambda i: (i, 0)),
            pl.BlockSpec((_BLK, D), lambda i: (i, 0)),
            pl.BlockSpec((D, D), lambda i: (0, 0)),
            pl.BlockSpec((D, D), lambda i: (0, 0)),
        ],
        out_specs=[
            pl.BlockSpec((_BLK, D), lambda i: (i, 0)),
            pl.BlockSpec((_BLK, D), lambda i: (i, 0)),
        ],
        out_shape=[
            jax.ShapeDtypeStruct((NP, D), jnp.float32),
            jax.ShapeDtypeStruct((NP, D), jnp.float32),
        ],
    )(x1, x2, w1, w2)


def _scale_body(xw1_ref, xw2_ref, deg1_ref, deg2_ref, y1_ref, y2_ref):
    d1 = lax.rsqrt(deg1_ref[...][:, 0:1] + 1.0)
    d2 = lax.rsqrt(deg2_ref[...][:, 0:1] + 1.0)
    y1_ref[...] = xw1_ref[...] * d1
    y2_ref[...] = xw2_ref[...] * d2


def _tc_scale(xw1, xw2, deg1, deg2):
    return pl.pallas_call(
        _scale_body,
        grid=(NP // _BLK,),
        in_specs=[
            pl.BlockSpec((_BLK, D), lambda i: (i, 0)),
            pl.BlockSpec((_BLK, D), lambda i: (i, 0)),
            pl.BlockSpec((_BLK, 16), lambda i: (i, 0)),
            pl.BlockSpec((_BLK, 16), lambda i: (i, 0)),
        ],
        out_specs=[
            pl.BlockSpec((_BLK, D), lambda i: (i, 0)),
            pl.BlockSpec((_BLK, D), lambda i: (i, 0)),
        ],
        out_shape=[
            jax.ShapeDtypeStruct((NP, D), jnp.float32),
            jax.ShapeDtypeStruct((NP, D), jnp.float32),
        ],
    )(xw1, xw2, deg1, deg2)


def _final_body(acc1_ref, y1_ref, deg1_ref, b1_ref,
                acc2_ref, y2_ref, deg2_ref, b2_ref,
                wp_ref, bp_ref, o1_ref, o2_ref):
    for acc, y, deg, b, o in (
        (acc1_ref, y1_ref, deg1_ref, b1_ref, o1_ref),
        (acc2_ref, y2_ref, deg2_ref, b2_ref, o2_ref),
    ):
        d = lax.rsqrt(deg[...][:, 0:1] + 1.0)
        t = (acc[...] + y[...]) * d + b[...]
        t = jnp.maximum(t, 0.0)
        o[...] = jnp.dot(t, wp_ref[...],
                         preferred_element_type=jnp.float32) + bp_ref[...]


def _tc_final(acc1, y1, deg1, b1, acc2, y2, deg2, b2, wp, bp):
    return pl.pallas_call(
        _final_body,
        grid=(NP // _BLK,),
        in_specs=[
            pl.BlockSpec((_BLK, D), lambda i: (i, 0)),
            pl.BlockSpec((_BLK, D), lambda i: (i, 0)),
            pl.BlockSpec((_BLK, 16), lambda i: (i, 0)),
            pl.BlockSpec((1, D), lambda i: (0, 0)),
            pl.BlockSpec((_BLK, D), lambda i: (i, 0)),
            pl.BlockSpec((_BLK, D), lambda i: (i, 0)),
            pl.BlockSpec((_BLK, 16), lambda i: (i, 0)),
            pl.BlockSpec((1, D), lambda i: (0, 0)),
            pl.BlockSpec((D, D), lambda i: (0, 0)),
            pl.BlockSpec((1, D), lambda i: (0, 0)),
        ],
        out_specs=[
            pl.BlockSpec((_BLK, D), lambda i: (i, 0)),
            pl.BlockSpec((_BLK, D), lambda i: (i, 0)),
        ],
        out_shape=[
            jax.ShapeDtypeStruct((NP, D), jnp.float32),
            jax.ShapeDtypeStruct((NP, D), jnp.float32),
        ],
    )(acc1, y1, deg1, b1, acc2, y2, deg2, b2, wp, bp)


def _prep_edges(ei):
    pad = EPAD - E
    src = jnp.concatenate([ei[0].astype(jnp.int32),
                           jnp.full((pad,), N, jnp.int32)])
    dst = jnp.concatenate([ei[1].astype(jnp.int32),
                           N + (jnp.arange(pad, dtype=jnp.int32) % (NP - N))])
    return (src.reshape(NS, NCHUNK, CHUNK), dst.reshape(NS, NCHUNK, CHUNK))


def kernel(x_item, x_user, edge_index_ui, edge_index_iu,
           W_ui, b_ui, W_iu, b_iu, W_proj, b_proj):
    f32 = jnp.float32
    xi = jnp.pad(x_item.astype(f32), ((0, NP - N), (0, 0)))
    xu = jnp.pad(x_user.astype(f32), ((0, NP - N), (0, 0)))
    src0, dst0 = _prep_edges(edge_index_ui)
    src1, dst1 = _prep_edges(edge_index_iu)
    dst_both = jnp.stack([dst0, dst1])

    zeros16 = jnp.zeros((ROWS_PT, 16), f32)
    zerosD = jnp.zeros((ROWS_PT, D), f32)

    deg = _deg_kernel(dst_both, zeros16)
    xw_ui, xw_iu = _tc_xw(xi, xu, W_ui, W_iu)
    y_ui, y_iu = _tc_scale(xw_ui, xw_iu, deg[0], deg[1])
    acc_ui = _edge_kernel(y_ui, src0, dst0, zerosD)
    acc_iu = _edge_kernel(y_iu, src1, dst1, zerosD)
    out_item, out_user = _tc_final(
        acc_ui, y_ui, deg[0], b_ui.reshape(1, D),
        acc_iu, y_iu, deg[1], b_iu.reshape(1, D),
        W_proj, b_proj.reshape(1, D))
    return (out_item[:N], out_user[:N])


# speedup vs baseline: 11.4291x; 1.2344x over previous
import functools

import jax
import jax.numpy as jnp
from jax import lax
from jax.experimental import pallas as pl
from jax.experimental.pallas import tpu as pltpu
from jax.experimental.pallas import tpu_sc as plsc

N = 10000
E = 320000
D = 128

NS = 16
NP = 10112
ROWS_PT = NP // NS
CHUNK = 64
IDXBLK = 32
NCHUNK = 320
DEPTH = 4
EPT = CHUNK * NCHUNK
EPAD = EPT * NS

_mesh2 = plsc.VectorSubcoreMesh(core_axis_name="c", subcore_axis_name="s")
_mesh1 = plsc.VectorSubcoreMesh(core_axis_name="c", subcore_axis_name="s",
                                num_cores=1)


@functools.partial(
    pl.kernel,
    mesh=_mesh2,
    compiler_params=pltpu.CompilerParams(use_tc_tiling_on_sc=False),
    out_type=jax.ShapeDtypeStruct((2, NP, 16), jnp.float32),
    scratch_types=[
        pltpu.VMEM_SHARED((NP, 16), jnp.float32),
        pltpu.VMEM((IDXBLK, CHUNK), jnp.int32),
        pltpu.VMEM((CHUNK, 16), jnp.float32),
        pltpu.SemaphoreType.DMA,
    ],
)
def _deg_kernel(dst_hbm, zeros_hbm, deg_out, deg_sh, dst_v, ones_v, sem):
    c = lax.axis_index("c")
    s = lax.axis_index("s")
    pltpu.sync_copy(zeros_hbm, deg_sh.at[pl.ds(s * ROWS_PT, ROWS_PT)])

    @pl.loop(0, CHUNK)
    def _(j):
        ones_v.at[j][...] = jnp.full((16,), 1.0, jnp.float32)

    plsc.subcore_barrier()

    @pl.loop(0, NCHUNK, step=IDXBLK)
    def _(r):
        pltpu.sync_copy(dst_hbm.at[c, s, pl.ds(r, IDXBLK)], dst_v)

        @pl.loop(0, IDXBLK)
        def _(j):
            pltpu.sync_copy(ones_v, deg_sh.at[dst_v.at[j]], add=True)

    plsc.subcore_barrier()
    pltpu.sync_copy(deg_sh.at[pl.ds(s * ROWS_PT, ROWS_PT)],
                    deg_out.at[c, pl.ds(s * ROWS_PT, ROWS_PT)])


@functools.partial(
    pl.kernel,
    mesh=_mesh1,
    out_type=jax.ShapeDtypeStruct((NP, D), jnp.float32),
    scratch_types=[
        pltpu.VMEM_SHARED((NP, D), jnp.float32),
        pltpu.VMEM((IDXBLK, CHUNK), jnp.int32),
        pltpu.VMEM((IDXBLK, CHUNK), jnp.int32),
        pltpu.VMEM((DEPTH * CHUNK, D), jnp.float32),
        pltpu.SemaphoreType.DMA,
        pltpu.SemaphoreType.DMA,
    ],
)
def _edge_kernel(y_hbm, src_hbm, dst_hbm, zeros_hbm, acc_out,
                 acc_sh, src_v, dst_v, rows_v, gsem, ssem):
    s = lax.axis_index("s")
    pltpu.sync_copy(zeros_hbm, acc_sh.at[pl.ds(s * ROWS_PT, ROWS_PT)])
    plsc.subcore_barrier()
    rows = [rows_v.at[pl.ds(b * CHUNK, CHUNK)] for b in range(DEPTH)]
    LAG = DEPTH - 1

    @pl.loop(0, NCHUNK, step=IDXBLK)
    def _(r):
        pltpu.sync_copy(src_hbm.at[s, pl.ds(r, IDXBLK)], src_v)
        pltpu.sync_copy(dst_hbm.at[s, pl.ds(r, IDXBLK)], dst_v)
        g = [None] * IDXBLK
        sc = [None] * IDXBLK
        for k in range(IDXBLK + LAG):
            if k < IDXBLK:
                if k >= DEPTH:
                    sc[k - DEPTH].wait()
                g[k] = pltpu.async_copy(y_hbm.at[src_v.at[k]],
                                        rows[k % DEPTH], gsem)
            j = k - LAG
            if j >= 0:
                g[j].wait()
                sc[j] = pltpu.async_copy(rows[j % DEPTH],
                                         acc_sh.at[dst_v.at[j]], ssem,
                                         add=True)
        for t in range(max(0, IDXBLK - DEPTH), IDXBLK):
            sc[t].wait()

    plsc.subcore_barrier()
    pltpu.sync_copy(acc_sh.at[pl.ds(s * ROWS_PT, ROWS_PT)],
                    acc_out.at[pl.ds(s * ROWS_PT, ROWS_PT)])


_BLK = 2528


def _xw_body(x1_ref, x2_ref, w1_ref, w2_ref, o1_ref, o2_ref):
    o1_ref[...] = jnp.dot(x1_ref[...], w1_ref[...],
                          preferred_element_type=jnp.float32)
    o2_ref[...] = jnp.dot(x2_ref[...], w2_ref[...],
                          preferred_element_type=jnp.float32)


def _tc_xw(x1, x2, w1, w2):
    return pl.pallas_call(
        _xw_body,
        grid=(NP // _BLK,),
        in_specs=[
            pl.BlockSpec((_BLK, D), lambda i: (i, 0)),
            pl.BlockSpec((_BLK, D), lambda i: (i, 0)),
            pl.BlockSpec((D, D), lambda i: (0, 0)),
            pl.BlockSpec((D, D), lambda i: (0, 0)),
        ],
        out_specs=[
            pl.BlockSpec((_BLK, D), lambda i: (i, 0)),
            pl.BlockSpec((_BLK, D), lambda i: (i, 0)),
        ],
        out_shape=[
            jax.ShapeDtypeStruct((NP, D), jnp.float32),
            jax.ShapeDtypeStruct((NP, D), jnp.float32),
        ],
    )(x1, x2, w1, w2)


def _scale_body(xw1_ref, xw2_ref, deg1_ref, deg2_ref, y1_ref, y2_ref):
    d1 = lax.rsqrt(deg1_ref[...][:, 0:1] + 1.0)
    d2 = lax.rsqrt(deg2_ref[...][:, 0:1] + 1.0)
    y1_ref[...] = xw1_ref[...] * d1
    y2_ref[...] = xw2_ref[...] * d2


def _tc_scale(xw1, xw2, deg1, deg2):
    return pl.pallas_call(
        _scale_body,
        grid=(NP // _BLK,),
        in_specs=[
            pl.BlockSpec((_BLK, D), lambda i: (i, 0)),
            pl.BlockSpec((_BLK, D), lambda i: (i, 0)),
            pl.BlockSpec((_BLK, 16), lambda i: (i, 0)),
            pl.BlockSpec((_BLK, 16), lambda i: (i, 0)),
        ],
        out_specs=[
            pl.BlockSpec((_BLK, D), lambda i: (i, 0)),
            pl.BlockSpec((_BLK, D), lambda i: (i, 0)),
        ],
        out_shape=[
            jax.ShapeDtypeStruct((NP, D), jnp.float32),
            jax.ShapeDtypeStruct((NP, D), jnp.float32),
        ],
    )(xw1, xw2, deg1, deg2)


def _final_body(acc1_ref, y1_ref, deg1_ref, b1_ref,
                acc2_ref, y2_ref, deg2_ref, b2_ref,
                wp_ref, bp_ref, o1_ref, o2_ref):
    for acc, y, deg, b, o in (
        (acc1_ref, y1_ref, deg1_ref, b1_ref, o1_ref),
        (acc2_ref, y2_ref, deg2_ref, b2_ref, o2_ref),
    ):
        d = lax.rsqrt(deg[...][:, 0:1] + 1.0)
        t = (acc[...] + y[...]) * d + b[...]
        t = jnp.maximum(t, 0.0)
        o[...] = jnp.dot(t, wp_ref[...],
                         preferred_element_type=jnp.float32) + bp_ref[...]


def _tc_final(acc1, y1, deg1, b1, acc2, y2, deg2, b2, wp, bp):
    return pl.pallas_call(
        _final_body,
        grid=(NP // _BLK,),
        in_specs=[
            pl.BlockSpec((_BLK, D), lambda i: (i, 0)),
            pl.BlockSpec((_BLK, D), lambda i: (i, 0)),
            pl.BlockSpec((_BLK, 16), lambda i: (i, 0)),
            pl.BlockSpec((1, D), lambda i: (0, 0)),
            pl.BlockSpec((_BLK, D), lambda i: (i, 0)),
            pl.BlockSpec((_BLK, D), lambda i: (i, 0)),
            pl.BlockSpec((_BLK, 16), lambda i: (i, 0)),
            pl.BlockSpec((1, D), lambda i: (0, 0)),
            pl.BlockSpec((D, D), lambda i: (0, 0)),
            pl.BlockSpec((1, D), lambda i: (0, 0)),
        ],
        out_specs=[
            pl.BlockSpec((_BLK, D), lambda i: (i, 0)),
            pl.BlockSpec((_BLK, D), lambda i: (i, 0)),
        ],
        out_shape=[
            jax.ShapeDtypeStruct((NP, D), jnp.float32),
            jax.ShapeDtypeStruct((NP, D), jnp.float32),
        ],
    )(acc1, y1, deg1, b1, acc2, y2, deg2, b2, wp, bp)


def _prep_edges(ei):
    pad = EPAD - E
    src = jnp.concatenate([ei[0].astype(jnp.int32),
                           jnp.full((pad,), N, jnp.int32)])
    dst = jnp.concatenate([ei[1].astype(jnp.int32),
                           N + (jnp.arange(pad, dtype=jnp.int32) % (NP - N))])
    return (src.reshape(NS, NCHUNK, CHUNK), dst.reshape(NS, NCHUNK, CHUNK))


def kernel(x_item, x_user, edge_index_ui, edge_index_iu,
           W_ui, b_ui, W_iu, b_iu, W_proj, b_proj):
    f32 = jnp.float32
    xi = jnp.pad(x_item.astype(f32), ((0, NP - N), (0, 0)))
    xu = jnp.pad(x_user.astype(f32), ((0, NP - N), (0, 0)))
    src0, dst0 = _prep_edges(edge_index_ui)
    src1, dst1 = _prep_edges(edge_index_iu)
    dst_both = jnp.stack([dst0, dst1])

    zeros16 = jnp.zeros((ROWS_PT, 16), f32)
    zerosD = jnp.zeros((ROWS_PT, D), f32)

    deg = _deg_kernel(dst_both, zeros16)
    xw_ui, xw_iu = _tc_xw(xi, xu, W_ui, W_iu)
    y_ui, y_iu = _tc_scale(xw_ui, xw_iu, deg[0], deg[1])
    acc_ui = _edge_kernel(y_ui, src0, dst0, zerosD)
    acc_iu = _edge_kernel(y_iu, src1, dst1, zerosD)
    out_item, out_user = _tc_final(
        acc_ui, y_ui, deg[0], b_ui.reshape(1, D),
        acc_iu, y_iu, deg[1], b_iu.reshape(1, D),
        W_proj, b_proj.reshape(1, D))
    return (out_item[:N], out_user[:N])


# speedup vs baseline: 26.0840x; 2.2822x over previous
import functools

import jax
import jax.numpy as jnp
from jax import lax
from jax.experimental import pallas as pl
from jax.experimental.pallas import tpu as pltpu
from jax.experimental.pallas import tpu_sc as plsc

N = 10000
E = 320000
D = 128

NS = 16
NP = 10112
ROWS_PT = NP // NS
CHUNK = 64
IDXBLK = 32
NCHUNK = 320
DEPTH = 4
EPT = CHUNK * NCHUNK
EPAD = EPT * NS

_mesh2 = plsc.VectorSubcoreMesh(core_axis_name="c", subcore_axis_name="s")
_mesh1 = plsc.VectorSubcoreMesh(core_axis_name="c", subcore_axis_name="s",
                                num_cores=1)


@functools.partial(
    pl.kernel,
    mesh=_mesh2,
    compiler_params=pltpu.CompilerParams(use_tc_tiling_on_sc=False),
    out_type=jax.ShapeDtypeStruct((2, NP, 16), jnp.float32),
    scratch_types=[
        pltpu.VMEM_SHARED((NP, 16), jnp.float32),
        pltpu.VMEM((IDXBLK, CHUNK), jnp.int32),
        pltpu.VMEM((CHUNK, 16), jnp.float32),
        pltpu.SemaphoreType.DMA,
    ],
)
def _deg_kernel(dst_hbm, zeros_hbm, deg_out, deg_sh, dst_v, ones_v, sem):
    c = lax.axis_index("c")
    s = lax.axis_index("s")
    pltpu.sync_copy(zeros_hbm, deg_sh.at[pl.ds(s * ROWS_PT, ROWS_PT)])

    @pl.loop(0, CHUNK)
    def _(j):
        ones_v.at[j][...] = jnp.full((16,), 1.0, jnp.float32)

    plsc.subcore_barrier()

    @pl.loop(0, NCHUNK, step=IDXBLK)
    def _(r):
        pltpu.sync_copy(dst_hbm.at[c, s, pl.ds(r, IDXBLK)], dst_v)

        @pl.loop(0, IDXBLK)
        def _(j):
            pltpu.sync_copy(ones_v, deg_sh.at[dst_v.at[j]], add=True)

    plsc.subcore_barrier()
    pltpu.sync_copy(deg_sh.at[pl.ds(s * ROWS_PT, ROWS_PT)],
                    deg_out.at[c, pl.ds(s * ROWS_PT, ROWS_PT)])


@functools.partial(
    pl.kernel,
    mesh=_mesh2,
    compiler_params=pltpu.CompilerParams(use_tc_tiling_on_sc=False),
    out_type=jax.ShapeDtypeStruct((2, NP, D), jnp.bfloat16),
    scratch_types=[
        pltpu.VMEM_SHARED((NP, D), jnp.bfloat16),
        pltpu.VMEM((IDXBLK, CHUNK), jnp.int32),
        pltpu.VMEM((IDXBLK, CHUNK), jnp.int32),
        pltpu.VMEM((DEPTH * CHUNK, D), jnp.bfloat16),
        pltpu.SemaphoreType.DMA,
        pltpu.SemaphoreType.DMA,
    ],
)
def _edge_kernel(y_hbm, src_hbm, dst_hbm, zeros_hbm, acc_out,
                 acc_sh, src_v, dst_v, rows_v, gsem, ssem):
    c = lax.axis_index("c")
    s = lax.axis_index("s")
    pltpu.sync_copy(zeros_hbm, acc_sh.at[pl.ds(s * ROWS_PT, ROWS_PT)])
    plsc.subcore_barrier()
    rows = [rows_v.at[pl.ds(b * CHUNK, CHUNK)] for b in range(DEPTH)]
    LAG = DEPTH - 1

    @pl.loop(0, NCHUNK, step=IDXBLK)
    def _(r):
        pltpu.sync_copy(src_hbm.at[c, s, pl.ds(r, IDXBLK)], src_v)
        pltpu.sync_copy(dst_hbm.at[c, s, pl.ds(r, IDXBLK)], dst_v)
        g = [None] * IDXBLK
        sc = [None] * IDXBLK
        for k in range(IDXBLK + LAG):
            if k < IDXBLK:
                if k >= DEPTH:
                    sc[k - DEPTH].wait()
                g[k] = pltpu.async_copy(y_hbm.at[c].at[src_v.at[k]],
                                        rows[k % DEPTH], gsem)
            j = k - LAG
            if j >= 0:
                g[j].wait()
                sc[j] = pltpu.async_copy(rows[j % DEPTH],
                                         acc_sh.at[dst_v.at[j]], ssem,
                                         add=True)
        for t in range(max(0, IDXBLK - DEPTH), IDXBLK):
            sc[t].wait()

    plsc.subcore_barrier()
    pltpu.sync_copy(acc_sh.at[pl.ds(s * ROWS_PT, ROWS_PT)],
                    acc_out.at[c, pl.ds(s * ROWS_PT, ROWS_PT)])


_BLK = 2528


def _xw_body(x1_ref, x2_ref, w1_ref, w2_ref, o1_ref, o2_ref):
    o1_ref[...] = jnp.dot(x1_ref[...], w1_ref[...],
                          preferred_element_type=jnp.float32)
    o2_ref[...] = jnp.dot(x2_ref[...], w2_ref[...],
                          preferred_element_type=jnp.float32)


def _tc_xw(x1, x2, w1, w2):
    return pl.pallas_call(
        _xw_body,
        grid=(NP // _BLK,),
        in_specs=[
            pl.BlockSpec((_BLK, D), lambda i: (i, 0)),
            pl.BlockSpec((_BLK, D), lambda i: (i, 0)),
            pl.BlockSpec((D, D), lambda i: (0, 0)),
            pl.BlockSpec((D, D), lambda i: (0, 0)),
        ],
        out_specs=[
            pl.BlockSpec((_BLK, D), lambda i: (i, 0)),
            pl.BlockSpec((_BLK, D), lambda i: (i, 0)),
        ],
        out_shape=[
            jax.ShapeDtypeStruct((NP, D), jnp.float32),
            jax.ShapeDtypeStruct((NP, D), jnp.float32),
        ],
    )(x1, x2, w1, w2)


def _scale_body(xw1_ref, xw2_ref, deg1_ref, deg2_ref,
                y1_ref, y2_ref, ybf_ref):
    d1 = lax.rsqrt(deg1_ref[...][:, 0:1] + 1.0)
    d2 = lax.rsqrt(deg2_ref[...][:, 0:1] + 1.0)
    v1 = xw1_ref[...] * d1
    v2 = xw2_ref[...] * d2
    y1_ref[...] = v1
    y2_ref[...] = v2
    ybf_ref[0] = v1.astype(jnp.bfloat16)
    ybf_ref[1] = v2.astype(jnp.bfloat16)


def _tc_scale(xw1, xw2, deg1, deg2):
    return pl.pallas_call(
        _scale_body,
        grid=(NP // _BLK,),
        in_specs=[
            pl.BlockSpec((_BLK, D), lambda i: (i, 0)),
            pl.BlockSpec((_BLK, D), lambda i: (i, 0)),
            pl.BlockSpec((_BLK, 16), lambda i: (i, 0)),
            pl.BlockSpec((_BLK, 16), lambda i: (i, 0)),
        ],
        out_specs=[
            pl.BlockSpec((_BLK, D), lambda i: (i, 0)),
            pl.BlockSpec((_BLK, D), lambda i: (i, 0)),
            pl.BlockSpec((2, _BLK, D), lambda i: (0, i, 0)),
        ],
        out_shape=[
            jax.ShapeDtypeStruct((NP, D), jnp.float32),
            jax.ShapeDtypeStruct((NP, D), jnp.float32),
            jax.ShapeDtypeStruct((2, NP, D), jnp.bfloat16),
        ],
    )(xw1, xw2, deg1, deg2)


def _final_body(acc1_ref, y1_ref, deg1_ref, b1_ref,
                acc2_ref, y2_ref, deg2_ref, b2_ref,
                wp_ref, bp_ref, o1_ref, o2_ref):
    for acc, y, deg, b, o in (
        (acc1_ref, y1_ref, deg1_ref, b1_ref, o1_ref),
        (acc2_ref, y2_ref, deg2_ref, b2_ref, o2_ref),
    ):
        d = lax.rsqrt(deg[...][:, 0:1] + 1.0)
        t = (acc[...].astype(jnp.float32) + y[...]) * d + b[...]
        t = jnp.maximum(t, 0.0)
        o[...] = jnp.dot(t, wp_ref[...],
                         preferred_element_type=jnp.float32) + bp_ref[...]


def _tc_final(acc1, y1, deg1, b1, acc2, y2, deg2, b2, wp, bp):
    return pl.pallas_call(
        _final_body,
        grid=(NP // _BLK,),
        in_specs=[
            pl.BlockSpec((_BLK, D), lambda i: (i, 0)),
            pl.BlockSpec((_BLK, D), lambda i: (i, 0)),
            pl.BlockSpec((_BLK, 16), lambda i: (i, 0)),
            pl.BlockSpec((1, D), lambda i: (0, 0)),
            pl.BlockSpec((_BLK, D), lambda i: (i, 0)),
            pl.BlockSpec((_BLK, D), lambda i: (i, 0)),
            pl.BlockSpec((_BLK, 16), lambda i: (i, 0)),
            pl.BlockSpec((1, D), lambda i: (0, 0)),
            pl.BlockSpec((D, D), lambda i: (0, 0)),
            pl.BlockSpec((1, D), lambda i: (0, 0)),
        ],
        out_specs=[
            pl.BlockSpec((_BLK, D), lambda i: (i, 0)),
            pl.BlockSpec((_BLK, D), lambda i: (i, 0)),
        ],
        out_shape=[
            jax.ShapeDtypeStruct((NP, D), jnp.float32),
            jax.ShapeDtypeStruct((NP, D), jnp.float32),
        ],
    )(acc1, y1, deg1, b1, acc2, y2, deg2, b2, wp, bp)


def _prep_edges(ei):
    pad = EPAD - E
    src = jnp.concatenate([ei[0].astype(jnp.int32),
                           jnp.full((pad,), N, jnp.int32)])
    dst = jnp.concatenate([ei[1].astype(jnp.int32),
                           N + (jnp.arange(pad, dtype=jnp.int32) % (NP - N))])
    return (src.reshape(NS, NCHUNK, CHUNK), dst.reshape(NS, NCHUNK, CHUNK))


def kernel(x_item, x_user, edge_index_ui, edge_index_iu,
           W_ui, b_ui, W_iu, b_iu, W_proj, b_proj):
    f32 = jnp.float32
    xi = jnp.pad(x_item.astype(f32), ((0, NP - N), (0, 0)))
    xu = jnp.pad(x_user.astype(f32), ((0, NP - N), (0, 0)))
    src0, dst0 = _prep_edges(edge_index_ui)
    src1, dst1 = _prep_edges(edge_index_iu)
    src_both = jnp.stack([src0, src1])
    dst_both = jnp.stack([dst0, dst1])

    zeros16 = jnp.zeros((ROWS_PT, 16), f32)
    zerosDbf = jnp.zeros((ROWS_PT, D), jnp.bfloat16)

    deg = _deg_kernel(dst_both, zeros16)
    xw_ui, xw_iu = _tc_xw(xi, xu, W_ui, W_iu)
    y_ui, y_iu, y_bf = _tc_scale(xw_ui, xw_iu, deg[0], deg[1])
    acc = _edge_kernel(y_bf, src_both, dst_both, zerosDbf)
    out_item, out_user = _tc_final(
        acc[0], y_ui, deg[0], b_ui.reshape(1, D),
        acc[1], y_iu, deg[1], b_iu.reshape(1, D),
        W_proj, b_proj.reshape(1, D))
    return (out_item[:N], out_user[:N])


# speedup vs baseline: 26.1698x; 1.0033x over previous
import functools

import jax
import jax.numpy as jnp
from jax import lax
from jax.experimental import pallas as pl
from jax.experimental.pallas import tpu as pltpu
from jax.experimental.pallas import tpu_sc as plsc

N = 10000
E = 320000
D = 128

NS = 16
NP = 10112
ROWS_PT = NP // NS
CHUNK = 64
IDXBLK = 32
NCHUNK = 320
DEPTH = 8
EPT = CHUNK * NCHUNK
EPAD = EPT * NS

_mesh2 = plsc.VectorSubcoreMesh(core_axis_name="c", subcore_axis_name="s")
_mesh1 = plsc.VectorSubcoreMesh(core_axis_name="c", subcore_axis_name="s",
                                num_cores=1)


@functools.partial(
    pl.kernel,
    mesh=_mesh2,
    compiler_params=pltpu.CompilerParams(use_tc_tiling_on_sc=False),
    out_type=jax.ShapeDtypeStruct((2, NP, 16), jnp.float32),
    scratch_types=[
        pltpu.VMEM_SHARED((NP, 16), jnp.float32),
        pltpu.VMEM((IDXBLK, CHUNK), jnp.int32),
        pltpu.VMEM((CHUNK, 16), jnp.float32),
        pltpu.SemaphoreType.DMA,
    ],
)
def _deg_kernel(dst_hbm, zeros_hbm, deg_out, deg_sh, dst_v, ones_v, sem):
    c = lax.axis_index("c")
    s = lax.axis_index("s")
    pltpu.sync_copy(zeros_hbm, deg_sh.at[pl.ds(s * ROWS_PT, ROWS_PT)])

    @pl.loop(0, CHUNK)
    def _(j):
        ones_v.at[j][...] = jnp.full((16,), 1.0, jnp.float32)

    plsc.subcore_barrier()

    @pl.loop(0, NCHUNK, step=IDXBLK)
    def _(r):
        pltpu.sync_copy(dst_hbm.at[c, s, pl.ds(r, IDXBLK)], dst_v)

        @pl.loop(0, IDXBLK)
        def _(j):
            pltpu.sync_copy(ones_v, deg_sh.at[dst_v.at[j]], add=True)

    plsc.subcore_barrier()
    pltpu.sync_copy(deg_sh.at[pl.ds(s * ROWS_PT, ROWS_PT)],
                    deg_out.at[c, pl.ds(s * ROWS_PT, ROWS_PT)])


@functools.partial(
    pl.kernel,
    mesh=_mesh2,
    compiler_params=pltpu.CompilerParams(use_tc_tiling_on_sc=False),
    out_type=jax.ShapeDtypeStruct((2, NP, D), jnp.bfloat16),
    scratch_types=[
        pltpu.VMEM_SHARED((NP, D), jnp.bfloat16),
        pltpu.VMEM((IDXBLK, CHUNK), jnp.int32),
        pltpu.VMEM((IDXBLK, CHUNK), jnp.int32),
        pltpu.VMEM((DEPTH * CHUNK, D), jnp.bfloat16),
        pltpu.SemaphoreType.DMA,
        pltpu.SemaphoreType.DMA,
    ],
)
def _edge_kernel(y_hbm, src_hbm, dst_hbm, zeros_hbm, acc_out,
                 acc_sh, src_v, dst_v, rows_v, gsem, ssem):
    c = lax.axis_index("c")
    s = lax.axis_index("s")
    pltpu.sync_copy(zeros_hbm, acc_sh.at[pl.ds(s * ROWS_PT, ROWS_PT)])
    plsc.subcore_barrier()
    rows = [rows_v.at[pl.ds(b * CHUNK, CHUNK)] for b in range(DEPTH)]
    LAG = DEPTH - 1

    @pl.loop(0, NCHUNK, step=IDXBLK)
    def _(r):
        pltpu.sync_copy(src_hbm.at[c, s, pl.ds(r, IDXBLK)], src_v)
        pltpu.sync_copy(dst_hbm.at[c, s, pl.ds(r, IDXBLK)], dst_v)
        g = [None] * IDXBLK
        sc = [None] * IDXBLK
        for k in range(IDXBLK + LAG):
            if k < IDXBLK:
                if k >= DEPTH:
                    sc[k - DEPTH].wait()
                g[k] = pltpu.async_copy(y_hbm.at[c].at[src_v.at[k]],
                                        rows[k % DEPTH], gsem)
            j = k - LAG
            if j >= 0:
                g[j].wait()
                sc[j] = pltpu.async_copy(rows[j % DEPTH],
                                         acc_sh.at[dst_v.at[j]], ssem,
                                         add=True)
        for t in range(max(0, IDXBLK - DEPTH), IDXBLK):
            sc[t].wait()

    plsc.subcore_barrier()
    pltpu.sync_copy(acc_sh.at[pl.ds(s * ROWS_PT, ROWS_PT)],
                    acc_out.at[c, pl.ds(s * ROWS_PT, ROWS_PT)])


_BLK = 2528


def _xw_body(x1_ref, x2_ref, w1_ref, w2_ref, o1_ref, o2_ref):
    o1_ref[...] = jnp.dot(x1_ref[...], w1_ref[...],
                          preferred_element_type=jnp.float32)
    o2_ref[...] = jnp.dot(x2_ref[...], w2_ref[...],
                          preferred_element_type=jnp.float32)


def _tc_xw(x1, x2, w1, w2):
    return pl.pallas_call(
        _xw_body,
        grid=(NP // _BLK,),
        in_specs=[
            pl.BlockSpec((_BLK, D), lambda i: (i, 0)),
            pl.BlockSpec((_BLK, D), lambda i: (i, 0)),
            pl.BlockSpec((D, D), lambda i: (0, 0)),
            pl.BlockSpec((D, D), lambda i: (0, 0)),
        ],
        out_specs=[
            pl.BlockSpec((_BLK, D), lambda i: (i, 0)),
            pl.BlockSpec((_BLK, D), lambda i: (i, 0)),
        ],
        out_shape=[
            jax.ShapeDtypeStruct((NP, D), jnp.float32),
            jax.ShapeDtypeStruct((NP, D), jnp.float32),
        ],
    )(x1, x2, w1, w2)


def _scale_body(xw1_ref, xw2_ref, deg1_ref, deg2_ref,
                y1_ref, y2_ref, ybf_ref):
    d1 = lax.rsqrt(deg1_ref[...][:, 0:1] + 1.0)
    d2 = lax.rsqrt(deg2_ref[...][:, 0:1] + 1.0)
    v1 = xw1_ref[...] * d1
    v2 = xw2_ref[...] * d2
    y1_ref[...] = v1
    y2_ref[...] = v2
    ybf_ref[0] = v1.astype(jnp.bfloat16)
    ybf_ref[1] = v2.astype(jnp.bfloat16)


def _tc_scale(xw1, xw2, deg1, deg2):
    return pl.pallas_call(
        _scale_body,
        grid=(NP // _BLK,),
        in_specs=[
            pl.BlockSpec((_BLK, D), lambda i: (i, 0)),
            pl.BlockSpec((_BLK, D), lambda i: (i, 0)),
            pl.BlockSpec((_BLK, 16), lambda i: (i, 0)),
            pl.BlockSpec((_BLK, 16), lambda i: (i, 0)),
        ],
        out_specs=[
            pl.BlockSpec((_BLK, D), lambda i: (i, 0)),
            pl.BlockSpec((_BLK, D), lambda i: (i, 0)),
            pl.BlockSpec((2, _BLK, D), lambda i: (0, i, 0)),
        ],
        out_shape=[
            jax.ShapeDtypeStruct((NP, D), jnp.float32),
            jax.ShapeDtypeStruct((NP, D), jnp.float32),
            jax.ShapeDtypeStruct((2, NP, D), jnp.bfloat16),
        ],
    )(xw1, xw2, deg1, deg2)


def _final_body(acc1_ref, y1_ref, deg1_ref, b1_ref,
                acc2_ref, y2_ref, deg2_ref, b2_ref,
                wp_ref, bp_ref, o1_ref, o2_ref):
    for acc, y, deg, b, o in (
        (acc1_ref, y1_ref, deg1_ref, b1_ref, o1_ref),
        (acc2_ref, y2_ref, deg2_ref, b2_ref, o2_ref),
    ):
        d = lax.rsqrt(deg[...][:, 0:1] + 1.0)
        t = (acc[...].astype(jnp.float32) + y[...]) * d + b[...]
        t = jnp.maximum(t, 0.0)
        o[...] = jnp.dot(t, wp_ref[...],
                         preferred_element_type=jnp.float32) + bp_ref[...]


def _tc_final(acc1, y1, deg1, b1, acc2, y2, deg2, b2, wp, bp):
    return pl.pallas_call(
        _final_body,
        grid=(NP // _BLK,),
        in_specs=[
            pl.BlockSpec((_BLK, D), lambda i: (i, 0)),
            pl.BlockSpec((_BLK, D), lambda i: (i, 0)),
            pl.BlockSpec((_BLK, 16), lambda i: (i, 0)),
            pl.BlockSpec((1, D), lambda i: (0, 0)),
            pl.BlockSpec((_BLK, D), lambda i: (i, 0)),
            pl.BlockSpec((_BLK, D), lambda i: (i, 0)),
            pl.BlockSpec((_BLK, 16), lambda i: (i, 0)),
            pl.BlockSpec((1, D), lambda i: (0, 0)),
            pl.BlockSpec((D, D), lambda i: (0, 0)),
            pl.BlockSpec((1, D), lambda i: (0, 0)),
        ],
        out_specs=[
            pl.BlockSpec((_BLK, D), lambda i: (i, 0)),
            pl.BlockSpec((_BLK, D), lambda i: (i, 0)),
        ],
        out_shape=[
            jax.ShapeDtypeStruct((NP, D), jnp.float32),
            jax.ShapeDtypeStruct((NP, D), jnp.float32),
        ],
    )(acc1, y1, deg1, b1, acc2, y2, deg2, b2, wp, bp)


def _prep_edges(ei):
    pad = EPAD - E
    src = jnp.concatenate([ei[0].astype(jnp.int32),
                           jnp.full((pad,), N, jnp.int32)])
    dst = jnp.concatenate([ei[1].astype(jnp.int32),
                           N + (jnp.arange(pad, dtype=jnp.int32) % (NP - N))])
    return (src.reshape(NS, NCHUNK, CHUNK), dst.reshape(NS, NCHUNK, CHUNK))


def kernel(x_item, x_user, edge_index_ui, edge_index_iu,
           W_ui, b_ui, W_iu, b_iu, W_proj, b_proj):
    f32 = jnp.float32
    xi = jnp.pad(x_item.astype(f32), ((0, NP - N), (0, 0)))
    xu = jnp.pad(x_user.astype(f32), ((0, NP - N), (0, 0)))
    src0, dst0 = _prep_edges(edge_index_ui)
    src1, dst1 = _prep_edges(edge_index_iu)
    src_both = jnp.stack([src0, src1])
    dst_both = jnp.stack([dst0, dst1])

    zeros16 = jnp.zeros((ROWS_PT, 16), f32)
    zerosDbf = jnp.zeros((ROWS_PT, D), jnp.bfloat16)

    deg = _deg_kernel(dst_both, zeros16)
    xw_ui, xw_iu = _tc_xw(xi, xu, W_ui, W_iu)
    y_ui, y_iu, y_bf = _tc_scale(xw_ui, xw_iu, deg[0], deg[1])
    acc = _edge_kernel(y_bf, src_both, dst_both, zerosDbf)
    out_item, out_user = _tc_final(
        acc[0], y_ui, deg[0], b_ui.reshape(1, D),
        acc[1], y_iu, deg[1], b_iu.reshape(1, D),
        W_proj, b_proj.reshape(1, D))
    return (out_item[:N], out_user[:N])


# speedup vs baseline: 27.1265x; 1.0366x over previous
import functools

import jax
import jax.numpy as jnp
from jax import lax
from jax.experimental import pallas as pl
from jax.experimental.pallas import tpu as pltpu
from jax.experimental.pallas import tpu_sc as plsc

N = 10000
E = 320000
D = 128

NS = 16
NP = 10112
ROWS_PT = NP // NS
CHUNK = 64
IDXBLK = 32
NCHUNK = 320
DEPTH = 8
EPT = CHUNK * NCHUNK
EPAD = EPT * NS

_mesh2 = plsc.VectorSubcoreMesh(core_axis_name="c", subcore_axis_name="s")
_mesh1 = plsc.VectorSubcoreMesh(core_axis_name="c", subcore_axis_name="s",
                                num_cores=1)


@functools.partial(
    pl.kernel,
    mesh=_mesh2,
    compiler_params=pltpu.CompilerParams(use_tc_tiling_on_sc=False),
    out_type=jax.ShapeDtypeStruct((2, NP, 16), jnp.float32),
    scratch_types=[
        pltpu.VMEM_SHARED((NP, 16), jnp.float32),
        pltpu.VMEM((IDXBLK, CHUNK), jnp.int32),
        pltpu.VMEM((CHUNK, 16), jnp.float32),
        pltpu.SemaphoreType.DMA,
    ],
)
def _deg_kernel(dst_hbm, zeros_hbm, deg_out, deg_sh, dst_v, ones_v, dsem):
    c = lax.axis_index("c")
    s = lax.axis_index("s")
    pltpu.sync_copy(zeros_hbm, deg_sh.at[pl.ds(s * ROWS_PT, ROWS_PT)])

    @pl.loop(0, CHUNK)
    def _(j):
        ones_v.at[j][...] = jnp.full((16,), 1.0, jnp.float32)

    plsc.subcore_barrier()

    @pl.loop(0, NCHUNK, step=IDXBLK)
    def _(r):
        pltpu.sync_copy(dst_hbm.at[c, s, pl.ds(r, IDXBLK)], dst_v)
        sc = [pltpu.async_copy(ones_v, deg_sh.at[dst_v.at[j]], dsem, add=True)
              for j in range(IDXBLK)]
        for cp in sc:
            cp.wait()

    plsc.subcore_barrier()
    pltpu.sync_copy(deg_sh.at[pl.ds(s * ROWS_PT, ROWS_PT)],
                    deg_out.at[c, pl.ds(s * ROWS_PT, ROWS_PT)])


@functools.partial(
    pl.kernel,
    mesh=_mesh2,
    compiler_params=pltpu.CompilerParams(use_tc_tiling_on_sc=False),
    out_type=jax.ShapeDtypeStruct((2, NP, D), jnp.bfloat16),
    scratch_types=[
        pltpu.VMEM_SHARED((NP, D), jnp.bfloat16),
        pltpu.VMEM((IDXBLK, CHUNK), jnp.int32),
        pltpu.VMEM((IDXBLK, CHUNK), jnp.int32),
        pltpu.VMEM((DEPTH * CHUNK, D), jnp.bfloat16),
        pltpu.SemaphoreType.DMA,
        pltpu.SemaphoreType.DMA,
    ],
)
def _edge_kernel(y_hbm, src_hbm, dst_hbm, zeros_hbm, acc_out,
                 acc_sh, src_v, dst_v, rows_v, gsem, ssem):
    c = lax.axis_index("c")
    s = lax.axis_index("s")
    pltpu.sync_copy(zeros_hbm, acc_sh.at[pl.ds(s * ROWS_PT, ROWS_PT)])
    plsc.subcore_barrier()
    rows = [rows_v.at[pl.ds(b * CHUNK, CHUNK)] for b in range(DEPTH)]
    LAG = DEPTH - 1

    @pl.loop(0, NCHUNK, step=IDXBLK)
    def _(r):
        pltpu.sync_copy(src_hbm.at[c, s, pl.ds(r, IDXBLK)], src_v)
        pltpu.sync_copy(dst_hbm.at[c, s, pl.ds(r, IDXBLK)], dst_v)
        g = [None] * IDXBLK
        sc = [None] * IDXBLK
        for k in range(IDXBLK + LAG):
            if k < IDXBLK:
                if k >= DEPTH:
                    sc[k - DEPTH].wait()
                g[k] = pltpu.async_copy(y_hbm.at[c].at[src_v.at[k]],
                                        rows[k % DEPTH], gsem)
            j = k - LAG
            if j >= 0:
                g[j].wait()
                sc[j] = pltpu.async_copy(rows[j % DEPTH],
                                         acc_sh.at[dst_v.at[j]], ssem,
                                         add=True)
        for t in range(max(0, IDXBLK - DEPTH), IDXBLK):
            sc[t].wait()

    plsc.subcore_barrier()
    pltpu.sync_copy(acc_sh.at[pl.ds(s * ROWS_PT, ROWS_PT)],
                    acc_out.at[c, pl.ds(s * ROWS_PT, ROWS_PT)])


_BLK = 2528


def _xw_body(x1_ref, x2_ref, w1_ref, w2_ref, o1_ref, o2_ref):
    o1_ref[...] = jnp.dot(x1_ref[...], w1_ref[...],
                          preferred_element_type=jnp.float32)
    o2_ref[...] = jnp.dot(x2_ref[...], w2_ref[...],
                          preferred_element_type=jnp.float32)


def _tc_xw(x1, x2, w1, w2):
    return pl.pallas_call(
        _xw_body,
        grid=(NP // _BLK,),
        in_specs=[
            pl.BlockSpec((_BLK, D), lambda i: (i, 0)),
            pl.BlockSpec((_BLK, D), lambda i: (i, 0)),
            pl.BlockSpec((D, D), lambda i: (0, 0)),
            pl.BlockSpec((D, D), lambda i: (0, 0)),
        ],
        out_specs=[
            pl.BlockSpec((_BLK, D), lambda i: (i, 0)),
            pl.BlockSpec((_BLK, D), lambda i: (i, 0)),
        ],
        out_shape=[
            jax.ShapeDtypeStruct((NP, D), jnp.float32),
            jax.ShapeDtypeStruct((NP, D), jnp.float32),
        ],
    )(x1, x2, w1, w2)


def _scale_body(xw1_ref, xw2_ref, deg1_ref, deg2_ref,
                y1_ref, y2_ref, ybf_ref):
    d1 = lax.rsqrt(deg1_ref[...][:, 0:1] + 1.0)
    d2 = lax.rsqrt(deg2_ref[...][:, 0:1] + 1.0)
    v1 = xw1_ref[...] * d1
    v2 = xw2_ref[...] * d2
    y1_ref[...] = v1
    y2_ref[...] = v2
    ybf_ref[0] = v1.astype(jnp.bfloat16)
    ybf_ref[1] = v2.astype(jnp.bfloat16)


def _tc_scale(xw1, xw2, deg1, deg2):
    return pl.pallas_call(
        _scale_body,
        grid=(NP // _BLK,),
        in_specs=[
            pl.BlockSpec((_BLK, D), lambda i: (i, 0)),
            pl.BlockSpec((_BLK, D), lambda i: (i, 0)),
            pl.BlockSpec((_BLK, 16), lambda i: (i, 0)),
            pl.BlockSpec((_BLK, 16), lambda i: (i, 0)),
        ],
        out_specs=[
            pl.BlockSpec((_BLK, D), lambda i: (i, 0)),
            pl.BlockSpec((_BLK, D), lambda i: (i, 0)),
            pl.BlockSpec((2, _BLK, D), lambda i: (0, i, 0)),
        ],
        out_shape=[
            jax.ShapeDtypeStruct((NP, D), jnp.float32),
            jax.ShapeDtypeStruct((NP, D), jnp.float32),
            jax.ShapeDtypeStruct((2, NP, D), jnp.bfloat16),
        ],
    )(xw1, xw2, deg1, deg2)


def _final_body(acc1_ref, y1_ref, deg1_ref, b1_ref,
                acc2_ref, y2_ref, deg2_ref, b2_ref,
                wp_ref, bp_ref, o1_ref, o2_ref):
    for acc, y, deg, b, o in (
        (acc1_ref, y1_ref, deg1_ref, b1_ref, o1_ref),
        (acc2_ref, y2_ref, deg2_ref, b2_ref, o2_ref),
    ):
        d = lax.rsqrt(deg[...][:, 0:1] + 1.0)
        t = (acc[...].astype(jnp.float32) + y[...]) * d + b[...]
        t = jnp.maximum(t, 0.0)
        o[...] = jnp.dot(t, wp_ref[...],
                         preferred_element_type=jnp.float32) + bp_ref[...]


def _tc_final(acc1, y1, deg1, b1, acc2, y2, deg2, b2, wp, bp):
    return pl.pallas_call(
        _final_body,
        grid=(NP // _BLK,),
        in_specs=[
            pl.BlockSpec((_BLK, D), lambda i: (i, 0)),
            pl.BlockSpec((_BLK, D), lambda i: (i, 0)),
            pl.BlockSpec((_BLK, 16), lambda i: (i, 0)),
            pl.BlockSpec((1, D), lambda i: (0, 0)),
            pl.BlockSpec((_BLK, D), lambda i: (i, 0)),
            pl.BlockSpec((_BLK, D), lambda i: (i, 0)),
            pl.BlockSpec((_BLK, 16), lambda i: (i, 0)),
            pl.BlockSpec((1, D), lambda i: (0, 0)),
            pl.BlockSpec((D, D), lambda i: (0, 0)),
            pl.BlockSpec((1, D), lambda i: (0, 0)),
        ],
        out_specs=[
            pl.BlockSpec((_BLK, D), lambda i: (i, 0)),
            pl.BlockSpec((_BLK, D), lambda i: (i, 0)),
        ],
        out_shape=[
            jax.ShapeDtypeStruct((NP, D), jnp.float32),
            jax.ShapeDtypeStruct((NP, D), jnp.float32),
        ],
    )(acc1, y1, deg1, b1, acc2, y2, deg2, b2, wp, bp)


def _prep_edges(ei):
    pad = EPAD - E
    src = jnp.concatenate([ei[0].astype(jnp.int32),
                           jnp.full((pad,), N, jnp.int32)])
    dst = jnp.concatenate([ei[1].astype(jnp.int32),
                           N + (jnp.arange(pad, dtype=jnp.int32) % (NP - N))])
    return (src.reshape(NS, NCHUNK, CHUNK), dst.reshape(NS, NCHUNK, CHUNK))


def kernel(x_item, x_user, edge_index_ui, edge_index_iu,
           W_ui, b_ui, W_iu, b_iu, W_proj, b_proj):
    f32 = jnp.float32
    xi = jnp.pad(x_item.astype(f32), ((0, NP - N), (0, 0)))
    xu = jnp.pad(x_user.astype(f32), ((0, NP - N), (0, 0)))
    src0, dst0 = _prep_edges(edge_index_ui)
    src1, dst1 = _prep_edges(edge_index_iu)
    src_both = jnp.stack([src0, src1])
    dst_both = jnp.stack([dst0, dst1])

    zeros16 = jnp.zeros((ROWS_PT, 16), f32)
    zerosDbf = jnp.zeros((ROWS_PT, D), jnp.bfloat16)

    deg = _deg_kernel(dst_both, zeros16)
    xw_ui, xw_iu = _tc_xw(xi, xu, W_ui, W_iu)
    y_ui, y_iu, y_bf = _tc_scale(xw_ui, xw_iu, deg[0], deg[1])
    acc = _edge_kernel(y_bf, src_both, dst_both, zerosDbf)
    out_item, out_user = _tc_final(
        acc[0], y_ui, deg[0], b_ui.reshape(1, D),
        acc[1], y_iu, deg[1], b_iu.reshape(1, D),
        W_proj, b_proj.reshape(1, D))
    return (out_item[:N], out_user[:N])
